# Initial kernel scaffold; baseline (speedup 1.0000x reference)
import jax
import jax.numpy as jnp
from jax.experimental import pallas as pl


def kernel(verts, edges, l1_W0, l1_W1, l1_b, l2_W0, l2_W1, l2_b):
    raise NotImplementedError("write your pallas kernel here")


# speedup vs baseline: 22.4841x; 22.4841x over previous
import functools

import jax
import jax.numpy as jnp
from jax import lax
from jax.experimental import pallas as pl
from jax.experimental.pallas import tpu as pltpu
from jax.experimental.pallas import tpu_sc as plsc

NC = 2
NS = 16
NW = NC * NS
CH = 80
RBLK = 1000


def _make_hist(n, e):
    epw = e // NW
    mesh = plsc.VectorSubcoreMesh(core_axis_name="c", subcore_axis_name="s")

    @functools.partial(
        pl.kernel,
        out_type=jax.ShapeDtypeStruct((NW, n), jnp.float32),
        mesh=mesh,
        scratch_types=[
            pltpu.VMEM((epw,), jnp.int32),
            pltpu.VMEM((n,), jnp.float32),
        ],
        compiler_params=pltpu.CompilerParams(needs_layout_passes=False),
    )
    def hist_kernel(row_hbm, out_hbm, rowv, hist):
        wid = lax.axis_index("s") * NC + lax.axis_index("c")
        pltpu.sync_copy(row_hbm.at[pl.ds(wid * epw, epw)], rowv)
        z16 = jnp.zeros((16,), jnp.float32)

        def zbody(i, carry):
            hist[pl.ds(i * 16, 16)] = z16
            return carry

        lax.fori_loop(0, n // 16, zbody, 0)
        ones16 = jnp.ones((16,), jnp.float32)

        def body(i, carry):
            idx = rowv[pl.ds(i * 16, 16)]
            plsc.addupdate_scatter(hist, [idx], ones16)
            return carry

        lax.fori_loop(0, epw // 16, body, 0)
        pltpu.sync_copy(hist, out_hbm.at[wid])

    return hist_kernel


def _make_agg(n, e, w):
    epw = e // NW
    steps = epw // CH
    rows_pt = n // NS
    zr = 25
    mesh = plsc.VectorSubcoreMesh(core_axis_name="c", subcore_axis_name="s")

    @functools.partial(
        pl.kernel,
        out_type=jax.ShapeDtypeStruct((NC, n, w), jnp.float32),
        mesh=mesh,
        scratch_types=[
            pltpu.VMEM((steps, CH), jnp.int32),
            pltpu.VMEM((steps, CH), jnp.int32),
            pltpu.VMEM((CH, w), jnp.float32),
            pltpu.VMEM((zr, w), jnp.float32),
            pltpu.VMEM_SHARED((n, w), jnp.float32),
            pltpu.SemaphoreType.DMA,
        ],
        compiler_params=pltpu.CompilerParams(
            needs_layout_passes=False, use_tc_tiling_on_sc=False),
    )
    def agg_kernel(zt_hbm, col_hbm, row_hbm, out_hbm,
                   colv, rowv, buf, zbuf, acc, sem):
        c = lax.axis_index("c")
        s = lax.axis_index("s")
        wid = s * NC + c
        z16 = jnp.zeros((16,), jnp.float32)
        for r in range(zr):
            for k in range(w // 16):
                zbuf[r, pl.ds(k * 16, 16)] = z16
        base = s * rows_pt
        for r in range(rows_pt // zr):
            pltpu.sync_copy(zbuf, acc.at[pl.ds(base + r * zr, zr)])
        pltpu.sync_copy(col_hbm.at[pl.ds(wid * steps, steps)], colv)
        pltpu.sync_copy(row_hbm.at[pl.ds(wid * steps, steps)], rowv)
        plsc.subcore_barrier()

        def body(j, carry):
            pltpu.async_copy(zt_hbm.at[colv.at[j]], buf, sem).wait()
            pltpu.sync_copy(buf, acc.at[rowv.at[j]], add=True)
            return carry

        lax.fori_loop(0, steps, body, 0)
        plsc.subcore_barrier()
        pltpu.sync_copy(acc.at[pl.ds(s * rows_pt, rows_pt)],
                        out_hbm.at[c, pl.ds(s * rows_pt, rows_pt)])

    return agg_kernel


def _dinv_from_hist(h_blk):
    deg = jnp.sum(h_blk, axis=1, keepdims=True)
    return jnp.where(deg > 0, lax.rsqrt(jnp.maximum(deg, 1e-30)), 0.0)


def _tc_layer1(verts, histT, w0, w1, b1):
    n, d = verts.shape
    h1 = w0.shape[1]

    def body(v_ref, h_ref, w0_ref, w1_ref, b_ref, y_ref, zt_ref):
        dinv = _dinv_from_hist(h_ref[...])
        v = v_ref[...]
        y_ref[...] = (jnp.dot(v, w0_ref[...], preferred_element_type=jnp.float32)
                      + b_ref[...])
        zt_ref[...] = jnp.dot(v, w1_ref[...],
                              preferred_element_type=jnp.float32) * dinv

    return pl.pallas_call(
        body,
        grid=(n // RBLK,),
        in_specs=[
            pl.BlockSpec((RBLK, d), lambda i: (i, 0)),
            pl.BlockSpec((RBLK, NW), lambda i: (i, 0)),
            pl.BlockSpec((d, h1), lambda i: (0, 0)),
            pl.BlockSpec((d, h1), lambda i: (0, 0)),
            pl.BlockSpec((1, h1), lambda i: (0, 0)),
        ],
        out_specs=[
            pl.BlockSpec((RBLK, h1), lambda i: (i, 0)),
            pl.BlockSpec((RBLK, h1), lambda i: (i, 0)),
        ],
        out_shape=[
            jax.ShapeDtypeStruct((n, h1), jnp.float32),
            jax.ShapeDtypeStruct((n, h1), jnp.float32),
        ],
    )(verts, histT, w0, w1, b1)


def _tc_layer2(y1, agg1, histT, w0, w1, b2):
    n, h1 = y1.shape
    h2 = w0.shape[1]

    def body(y_ref, a_ref, h_ref, w0_ref, w1_ref, b_ref, y2_ref, zt2_ref):
        dinv = _dinv_from_hist(h_ref[...])
        a = (a_ref[0] + a_ref[1]) * dinv
        hid = jnp.maximum(y_ref[...] - a, 0.0)
        y2_ref[...] = (jnp.dot(hid, w0_ref[...],
                               preferred_element_type=jnp.float32) + b_ref[...])
        zt2_ref[...] = jnp.dot(hid, w1_ref[...],
                               preferred_element_type=jnp.float32) * dinv

    return pl.pallas_call(
        body,
        grid=(n // RBLK,),
        in_specs=[
            pl.BlockSpec((RBLK, h1), lambda i: (i, 0)),
            pl.BlockSpec((NC, RBLK, h1), lambda i: (0, i, 0)),
            pl.BlockSpec((RBLK, NW), lambda i: (i, 0)),
            pl.BlockSpec((h1, h2), lambda i: (0, 0)),
            pl.BlockSpec((h1, h2), lambda i: (0, 0)),
            pl.BlockSpec((1, h2), lambda i: (0, 0)),
        ],
        out_specs=[
            pl.BlockSpec((RBLK, h2), lambda i: (i, 0)),
            pl.BlockSpec((RBLK, h2), lambda i: (i, 0)),
        ],
        out_shape=[
            jax.ShapeDtypeStruct((n, h2), jnp.float32),
            jax.ShapeDtypeStruct((n, h2), jnp.float32),
        ],
    )(y1, agg1, histT, w0, w1, b2)


def _tc_final(y2, agg2, histT):
    n, h2 = y2.shape

    def body(y_ref, a_ref, h_ref, o_ref):
        dinv = _dinv_from_hist(h_ref[...])
        o_ref[...] = y_ref[...] - (a_ref[0] + a_ref[1]) * dinv

    return pl.pallas_call(
        body,
        grid=(n // RBLK,),
        in_specs=[
            pl.BlockSpec((RBLK, h2), lambda i: (i, 0)),
            pl.BlockSpec((NC, RBLK, h2), lambda i: (0, i, 0)),
            pl.BlockSpec((RBLK, NW), lambda i: (i, 0)),
        ],
        out_specs=pl.BlockSpec((RBLK, h2), lambda i: (i, 0)),
        out_shape=jax.ShapeDtypeStruct((n, h2), jnp.float32),
    )(y2, agg2, histT)


def kernel(verts, edges, l1_W0, l1_W1, l1_b, l2_W0, l2_W1, l2_b):
    n, _ = verts.shape
    e = edges.shape[0]
    row = edges[:, 0]
    col = edges[:, 1]
    row2d = row.reshape(e // CH, CH)
    col2d = col.reshape(e // CH, CH)

    hist = _make_hist(n, e)(row)
    histT = hist.T

    y1, zt1 = _tc_layer1(verts, histT, l1_W0, l1_W1, l1_b.reshape(1, -1))
    agg1 = _make_agg(n, e, l1_W0.shape[1])(zt1, col2d, row2d)
    y2, zt2 = _tc_layer2(y1, agg1, histT, l2_W0, l2_W1, l2_b.reshape(1, -1))
    agg2 = _make_agg(n, e, l2_W0.shape[1])(zt2, col2d, row2d)
    return _tc_final(y2, agg2, histT)


# speedup vs baseline: 33.6978x; 1.4987x over previous
import functools

import jax
import jax.numpy as jnp
from jax import lax
from jax.experimental import pallas as pl
from jax.experimental.pallas import tpu as pltpu
from jax.experimental.pallas import tpu_sc as plsc

NC = 2
NS = 16
NW = NC * NS
CH = 100
RBLK = 1000


def _make_hist(n, e):
    epw = e // NW
    mesh = plsc.VectorSubcoreMesh(core_axis_name="c", subcore_axis_name="s")

    @functools.partial(
        pl.kernel,
        out_type=jax.ShapeDtypeStruct((NW, n), jnp.float32),
        mesh=mesh,
        scratch_types=[
            pltpu.VMEM((epw,), jnp.int32),
            pltpu.VMEM((n,), jnp.float32),
        ],
        compiler_params=pltpu.CompilerParams(needs_layout_passes=False),
    )
    def hist_kernel(row_hbm, out_hbm, rowv, hist):
        wid = lax.axis_index("s") * NC + lax.axis_index("c")
        pltpu.sync_copy(row_hbm.at[pl.ds(wid * epw, epw)], rowv)
        z16 = jnp.zeros((16,), jnp.float32)

        def zbody(i, carry):
            hist[pl.ds(i * 16, 16)] = z16
            return carry

        lax.fori_loop(0, n // 16, zbody, 0)
        ones16 = jnp.ones((16,), jnp.float32)

        def body(i, carry):
            idx = rowv[pl.ds(i * 16, 16)]
            plsc.addupdate_scatter(hist, [idx], ones16)
            return carry

        lax.fori_loop(0, epw // 16, body, 0)
        pltpu.sync_copy(hist, out_hbm.at[wid])

    return hist_kernel


def _make_agg(n, e, w):
    epw = e // NW
    steps = epw // CH
    rows_pt = n // NS
    zr = 25
    mesh = plsc.VectorSubcoreMesh(core_axis_name="c", subcore_axis_name="s")

    @functools.partial(
        pl.kernel,
        out_type=jax.ShapeDtypeStruct((NC, n, w), jnp.float32),
        mesh=mesh,
        scratch_types=[
            pltpu.VMEM((steps, CH), jnp.int32),
            pltpu.VMEM((steps, CH), jnp.int32),
            pltpu.VMEM((CH, w), jnp.float32),
            pltpu.VMEM((CH, w), jnp.float32),
            pltpu.VMEM((zr, w), jnp.float32),
            pltpu.VMEM_SHARED((n, w), jnp.float32),
            pltpu.SemaphoreType.DMA,
            pltpu.SemaphoreType.DMA,
        ],
        compiler_params=pltpu.CompilerParams(
            needs_layout_passes=False, use_tc_tiling_on_sc=False),
    )
    def agg_kernel(zt_hbm, col_hbm, row_hbm, out_hbm,
                   colv, rowv, buf0, buf1, zbuf, acc, sem0, sem1):
        c = lax.axis_index("c")
        s = lax.axis_index("s")
        wid = s * NC + c
        z16 = jnp.zeros((16,), jnp.float32)
        for r in range(zr):
            for k in range(w // 16):
                zbuf[r, pl.ds(k * 16, 16)] = z16
        base = s * rows_pt
        for r in range(rows_pt // zr):
            pltpu.sync_copy(zbuf, acc.at[pl.ds(base + r * zr, zr)])
        pltpu.sync_copy(col_hbm.at[pl.ds(wid * steps, steps)], colv)
        pltpu.sync_copy(row_hbm.at[pl.ds(wid * steps, steps)], rowv)
        plsc.subcore_barrier()

        g0 = pltpu.async_copy(zt_hbm.at[colv.at[0]], buf0, sem0)

        def body(i, carry):
            j0 = 2 * i
            pltpu.async_copy(zt_hbm.at[colv.at[j0 + 1]], buf1, sem1)
            pltpu.make_async_copy(zt_hbm.at[colv.at[j0]], buf0, sem0).wait()
            pltpu.sync_copy(buf0, acc.at[rowv.at[j0]], add=True)

            @pl.when(j0 + 2 < steps)
            def _():
                pltpu.async_copy(zt_hbm.at[colv.at[j0 + 2]], buf0, sem0)

            pltpu.make_async_copy(zt_hbm.at[colv.at[j0 + 1]], buf1, sem1).wait()
            pltpu.sync_copy(buf1, acc.at[rowv.at[j0 + 1]], add=True)
            return carry

        lax.fori_loop(0, steps // 2, body, 0)
        plsc.subcore_barrier()
        pltpu.sync_copy(acc.at[pl.ds(s * rows_pt, rows_pt)],
                        out_hbm.at[c, pl.ds(s * rows_pt, rows_pt)])

    return agg_kernel


def _dinv_from_hist(h_blk):
    deg = jnp.sum(h_blk, axis=1, keepdims=True)
    return jnp.where(deg > 0, lax.rsqrt(jnp.maximum(deg, 1e-30)), 0.0)


def _tc_layer1(verts, histT, w0, w1, b1):
    n, d = verts.shape
    h1 = w0.shape[1]

    def body(v_ref, h_ref, w0_ref, w1_ref, b_ref, y_ref, zt_ref):
        dinv = _dinv_from_hist(h_ref[...])
        v = v_ref[...]
        y_ref[...] = (jnp.dot(v, w0_ref[...], preferred_element_type=jnp.float32)
                      + b_ref[...])
        zt_ref[...] = jnp.dot(v, w1_ref[...],
                              preferred_element_type=jnp.float32) * dinv

    return pl.pallas_call(
        body,
        grid=(n // RBLK,),
        in_specs=[
            pl.BlockSpec((RBLK, d), lambda i: (i, 0)),
            pl.BlockSpec((RBLK, NW), lambda i: (i, 0)),
            pl.BlockSpec((d, h1), lambda i: (0, 0)),
            pl.BlockSpec((d, h1), lambda i: (0, 0)),
            pl.BlockSpec((1, h1), lambda i: (0, 0)),
        ],
        out_specs=[
            pl.BlockSpec((RBLK, h1), lambda i: (i, 0)),
            pl.BlockSpec((RBLK, h1), lambda i: (i, 0)),
        ],
        out_shape=[
            jax.ShapeDtypeStruct((n, h1), jnp.float32),
            jax.ShapeDtypeStruct((n, h1), jnp.float32),
        ],
    )(verts, histT, w0, w1, b1)


def _tc_layer2(y1, agg1, histT, w0, w1, b2):
    n, h1 = y1.shape
    h2 = w0.shape[1]

    def body(y_ref, a_ref, h_ref, w0_ref, w1_ref, b_ref, y2_ref, zt2_ref):
        dinv = _dinv_from_hist(h_ref[...])
        a = (a_ref[0] + a_ref[1]) * dinv
        hid = jnp.maximum(y_ref[...] - a, 0.0)
        y2_ref[...] = (jnp.dot(hid, w0_ref[...],
                               preferred_element_type=jnp.float32) + b_ref[...])
        zt2_ref[...] = jnp.dot(hid, w1_ref[...],
                               preferred_element_type=jnp.float32) * dinv

    return pl.pallas_call(
        body,
        grid=(n // RBLK,),
        in_specs=[
            pl.BlockSpec((RBLK, h1), lambda i: (i, 0)),
            pl.BlockSpec((NC, RBLK, h1), lambda i: (0, i, 0)),
            pl.BlockSpec((RBLK, NW), lambda i: (i, 0)),
            pl.BlockSpec((h1, h2), lambda i: (0, 0)),
            pl.BlockSpec((h1, h2), lambda i: (0, 0)),
            pl.BlockSpec((1, h2), lambda i: (0, 0)),
        ],
        out_specs=[
            pl.BlockSpec((RBLK, h2), lambda i: (i, 0)),
            pl.BlockSpec((RBLK, h2), lambda i: (i, 0)),
        ],
        out_shape=[
            jax.ShapeDtypeStruct((n, h2), jnp.float32),
            jax.ShapeDtypeStruct((n, h2), jnp.float32),
        ],
    )(y1, agg1, histT, w0, w1, b2)


def _tc_final(y2, agg2, histT):
    n, h2 = y2.shape

    def body(y_ref, a_ref, h_ref, o_ref):
        dinv = _dinv_from_hist(h_ref[...])
        o_ref[...] = y_ref[...] - (a_ref[0] + a_ref[1]) * dinv

    return pl.pallas_call(
        body,
        grid=(n // RBLK,),
        in_specs=[
            pl.BlockSpec((RBLK, h2), lambda i: (i, 0)),
            pl.BlockSpec((NC, RBLK, h2), lambda i: (0, i, 0)),
            pl.BlockSpec((RBLK, NW), lambda i: (i, 0)),
        ],
        out_specs=pl.BlockSpec((RBLK, h2), lambda i: (i, 0)),
        out_shape=jax.ShapeDtypeStruct((n, h2), jnp.float32),
    )(y2, agg2, histT)


def kernel(verts, edges, l1_W0, l1_W1, l1_b, l2_W0, l2_W1, l2_b):
    n, _ = verts.shape
    e = edges.shape[0]
    row = edges[:, 0]
    col = edges[:, 1]
    row2d = row.reshape(e // CH, CH)
    col2d = col.reshape(e // CH, CH)

    hist = _make_hist(n, e)(row)
    histT = hist.T

    y1, zt1 = _tc_layer1(verts, histT, l1_W0, l1_W1, l1_b.reshape(1, -1))
    agg1 = _make_agg(n, e, l1_W0.shape[1])(zt1, col2d, row2d)
    y2, zt2 = _tc_layer2(y1, agg1, histT, l2_W0, l2_W1, l2_b.reshape(1, -1))
    agg2 = _make_agg(n, e, l2_W0.shape[1])(zt2, col2d, row2d)
    return _tc_final(y2, agg2, histT)


# speedup vs baseline: 34.1779x; 1.0142x over previous
import functools

import jax
import jax.numpy as jnp
from jax import lax
from jax.experimental import pallas as pl
from jax.experimental.pallas import tpu as pltpu
from jax.experimental.pallas import tpu_sc as plsc

NC = 2
NS = 16
NW = NC * NS
CH = 100
RBLK = 1000


def _make_hist(n, e):
    epw = e // NW
    mesh = plsc.VectorSubcoreMesh(core_axis_name="c", subcore_axis_name="s")

    @functools.partial(
        pl.kernel,
        out_type=jax.ShapeDtypeStruct((NW, n), jnp.float32),
        mesh=mesh,
        scratch_types=[
            pltpu.VMEM((epw,), jnp.int32),
            pltpu.VMEM((n,), jnp.float32),
        ],
        compiler_params=pltpu.CompilerParams(needs_layout_passes=False),
    )
    def hist_kernel(row_hbm, out_hbm, rowv, hist):
        wid = lax.axis_index("s") * NC + lax.axis_index("c")
        pltpu.sync_copy(row_hbm.at[pl.ds(wid * epw, epw)], rowv)
        z16 = jnp.zeros((16,), jnp.float32)

        def zbody(i, carry):
            hist[pl.ds(i * 16, 16)] = z16
            return carry

        lax.fori_loop(0, n // 16, zbody, 0)
        ones16 = jnp.ones((16,), jnp.float32)

        def body(i, carry):
            idx = rowv[pl.ds(i * 16, 16)]
            plsc.addupdate_scatter(hist, [idx], ones16)
            return carry

        lax.fori_loop(0, epw // 16, body, 0)
        pltpu.sync_copy(hist, out_hbm.at[wid])

    return hist_kernel


def _make_agg(n, e, w):
    epw = e // NW
    steps = epw // CH
    rows_pt = n // NS
    zr = 25
    mesh = plsc.VectorSubcoreMesh(core_axis_name="c", subcore_axis_name="s")

    @functools.partial(
        pl.kernel,
        out_type=jax.ShapeDtypeStruct((NC, n, w), jnp.float32),
        mesh=mesh,
        scratch_types=[
            pltpu.VMEM((steps, CH), jnp.int32),
            pltpu.VMEM((steps, CH), jnp.int32),
            pltpu.VMEM((CH, w), jnp.float32),
            pltpu.VMEM((CH, w), jnp.float32),
            pltpu.VMEM((zr, w), jnp.float32),
            pltpu.VMEM_SHARED((n, w), jnp.float32),
            pltpu.SemaphoreType.DMA,
            pltpu.SemaphoreType.DMA,
        ],
        compiler_params=pltpu.CompilerParams(
            needs_layout_passes=False, use_tc_tiling_on_sc=False),
    )
    def agg_kernel(zt_hbm, col_hbm, row_hbm, out_hbm,
                   colv, rowv, buf0, buf1, zbuf, acc, sem0, sem1):
        c = lax.axis_index("c")
        s = lax.axis_index("s")
        wid = s * NC + c
        z16 = jnp.zeros((16,), jnp.float32)
        for r in range(zr):
            for k in range(w // 16):
                zbuf[r, pl.ds(k * 16, 16)] = z16
        base = s * rows_pt
        for r in range(rows_pt // zr):
            pltpu.sync_copy(zbuf, acc.at[pl.ds(base + r * zr, zr)])
        pltpu.sync_copy(col_hbm.at[pl.ds(wid * steps, steps)], colv)
        pltpu.sync_copy(row_hbm.at[pl.ds(wid * steps, steps)], rowv)
        plsc.subcore_barrier()

        g0 = pltpu.async_copy(zt_hbm.at[colv.at[0]], buf0, sem0)

        def body(i, carry):
            j0 = 2 * i
            pltpu.async_copy(zt_hbm.at[colv.at[j0 + 1]], buf1, sem1)
            pltpu.make_async_copy(zt_hbm.at[colv.at[j0]], buf0, sem0).wait()
            pltpu.sync_copy(buf0, acc.at[rowv.at[j0]], add=True)

            @pl.when(j0 + 2 < steps)
            def _():
                pltpu.async_copy(zt_hbm.at[colv.at[j0 + 2]], buf0, sem0)

            pltpu.make_async_copy(zt_hbm.at[colv.at[j0 + 1]], buf1, sem1).wait()
            pltpu.sync_copy(buf1, acc.at[rowv.at[j0 + 1]], add=True)
            return carry

        lax.fori_loop(0, steps // 2, body, 0)
        plsc.subcore_barrier()
        pltpu.sync_copy(acc.at[pl.ds(s * rows_pt, rows_pt)],
                        out_hbm.at[c, pl.ds(s * rows_pt, rows_pt)])

    return agg_kernel


def _dinv_from_hist(h_blk):
    ones = jnp.ones((NW, 1), jnp.float32)
    deg = lax.dot_general(h_blk, ones, (((0,), (0,)), ((), ())),
                          precision=lax.Precision.HIGHEST,
                          preferred_element_type=jnp.float32)
    return jnp.where(deg > 0, lax.rsqrt(jnp.maximum(deg, 1e-30)), 0.0)


def _full(shape):
    nd = len(shape)
    return pl.BlockSpec(shape, lambda: (0,) * nd)


def _tc_layer1(verts, hist, w0, w1, b1):
    n, d = verts.shape
    h1 = w0.shape[1]

    def body(v_ref, h_ref, w0_ref, w1_ref, b_ref, y_ref, zt_ref):
        dinv = _dinv_from_hist(h_ref[...])
        v = v_ref[...]
        y_ref[...] = (jnp.dot(v, w0_ref[...], preferred_element_type=jnp.float32)
                      + b_ref[...])
        zt_ref[...] = jnp.dot(v, w1_ref[...],
                              preferred_element_type=jnp.float32) * dinv

    return pl.pallas_call(
        body,
        in_specs=[_full((n, d)), _full((NW, n)), _full((d, h1)),
                  _full((d, h1)), _full((1, h1))],
        out_specs=[_full((n, h1)), _full((n, h1))],
        out_shape=[
            jax.ShapeDtypeStruct((n, h1), jnp.float32),
            jax.ShapeDtypeStruct((n, h1), jnp.float32),
        ],
    )(verts, hist, w0, w1, b1)


def _tc_layer2(y1, agg1, hist, w0, w1, b2):
    n, h1 = y1.shape
    h2 = w0.shape[1]

    def body(y_ref, a_ref, h_ref, w0_ref, w1_ref, b_ref, y2_ref, zt2_ref):
        dinv = _dinv_from_hist(h_ref[...])
        a = (a_ref[0] + a_ref[1]) * dinv
        hid = jnp.maximum(y_ref[...] - a, 0.0)
        y2_ref[...] = (jnp.dot(hid, w0_ref[...],
                               preferred_element_type=jnp.float32) + b_ref[...])
        zt2_ref[...] = jnp.dot(hid, w1_ref[...],
                               preferred_element_type=jnp.float32) * dinv

    return pl.pallas_call(
        body,
        in_specs=[_full((n, h1)), _full((NC, n, h1)), _full((NW, n)),
                  _full((h1, h2)), _full((h1, h2)), _full((1, h2))],
        out_specs=[_full((n, h2)), _full((n, h2))],
        out_shape=[
            jax.ShapeDtypeStruct((n, h2), jnp.float32),
            jax.ShapeDtypeStruct((n, h2), jnp.float32),
        ],
    )(y1, agg1, hist, w0, w1, b2)


def _tc_final(y2, agg2, hist):
    n, h2 = y2.shape

    def body(y_ref, a_ref, h_ref, o_ref):
        dinv = _dinv_from_hist(h_ref[...])
        o_ref[...] = y_ref[...] - (a_ref[0] + a_ref[1]) * dinv

    return pl.pallas_call(
        body,
        in_specs=[_full((n, h2)), _full((NC, n, h2)), _full((NW, n))],
        out_specs=_full((n, h2)),
        out_shape=jax.ShapeDtypeStruct((n, h2), jnp.float32),
    )(y2, agg2, hist)


def kernel(verts, edges, l1_W0, l1_W1, l1_b, l2_W0, l2_W1, l2_b):
    n, _ = verts.shape
    e = edges.shape[0]
    row = edges[:, 0]
    col = edges[:, 1]
    row2d = row.reshape(e // CH, CH)
    col2d = col.reshape(e // CH, CH)

    hist = _make_hist(n, e)(row)

    y1, zt1 = _tc_layer1(verts, hist, l1_W0, l1_W1, l1_b.reshape(1, -1))
    agg1 = _make_agg(n, e, l1_W0.shape[1])(zt1, col2d, row2d)
    y2, zt2 = _tc_layer2(y1, agg1, hist, l2_W0, l2_W1, l2_b.reshape(1, -1))
    agg2 = _make_agg(n, e, l2_W0.shape[1])(zt2, col2d, row2d)
    return _tc_final(y2, agg2, hist)


# speedup vs baseline: 42.7645x; 1.2512x over previous
import functools

import jax
import jax.numpy as jnp
from jax import lax
from jax.experimental import pallas as pl
from jax.experimental.pallas import tpu as pltpu
from jax.experimental.pallas import tpu_sc as plsc

NC = 2
NS = 16
NW = NC * NS
CH = 125
GB = 4
RBLK = 1000


def _make_hist(n, e):
    epw = e // NW
    mesh = plsc.VectorSubcoreMesh(core_axis_name="c", subcore_axis_name="s")

    @functools.partial(
        pl.kernel,
        out_type=jax.ShapeDtypeStruct((NW, n), jnp.float32),
        mesh=mesh,
        scratch_types=[
            pltpu.VMEM((epw,), jnp.int32),
            pltpu.VMEM((n,), jnp.float32),
        ],
        compiler_params=pltpu.CompilerParams(needs_layout_passes=False),
    )
    def hist_kernel(row_hbm, out_hbm, rowv, hist):
        wid = lax.axis_index("s") * NC + lax.axis_index("c")
        pltpu.sync_copy(row_hbm.at[pl.ds(wid * epw, epw)], rowv)
        z16 = jnp.zeros((16,), jnp.float32)

        def zbody(i, carry):
            hist[pl.ds(i * 16, 16)] = z16
            return carry

        lax.fori_loop(0, n // 16, zbody, 0)
        ones16 = jnp.ones((16,), jnp.float32)

        def body(i, carry):
            idx = rowv[pl.ds(i * 16, 16)]
            plsc.addupdate_scatter(hist, [idx], ones16)
            return carry

        lax.fori_loop(0, epw // 16, body, 0)
        pltpu.sync_copy(hist, out_hbm.at[wid])

    return hist_kernel


def _make_agg(n, e, w):
    epw = e // NW
    steps = epw // CH
    rows_pt = n // NS
    zr = 25
    mesh = plsc.VectorSubcoreMesh(core_axis_name="c", subcore_axis_name="s")

    @functools.partial(
        pl.kernel,
        out_type=jax.ShapeDtypeStruct((NC, n, w), jnp.float32),
        mesh=mesh,
        scratch_types=[
            pltpu.VMEM((steps, CH), jnp.int32),
            pltpu.VMEM((steps, CH), jnp.int32),
            [pltpu.VMEM((CH, w), jnp.float32) for _ in range(GB)],
            [pltpu.VMEM((CH, w), jnp.float32) for _ in range(GB)],
            pltpu.VMEM((zr, w), jnp.float32),
            pltpu.VMEM_SHARED((n, w), jnp.float32),
            pltpu.SemaphoreType.DMA,
            pltpu.SemaphoreType.DMA,
        ],
        compiler_params=pltpu.CompilerParams(
            needs_layout_passes=False, use_tc_tiling_on_sc=False),
    )
    def agg_kernel(zt_hbm, col_hbm, row_hbm, out_hbm,
                   colv, rowv, bufa, bufb, zbuf, acc, gsa, gsb):
        c = lax.axis_index("c")
        s = lax.axis_index("s")
        wid = s * NC + c
        z16 = jnp.zeros((16,), jnp.float32)
        for r in range(zr):
            for k in range(w // 16):
                zbuf[r, pl.ds(k * 16, 16)] = z16
        base = s * rows_pt
        for r in range(rows_pt // zr):
            pltpu.sync_copy(zbuf, acc.at[pl.ds(base + r * zr, zr)])
        pltpu.sync_copy(col_hbm.at[pl.ds(wid * steps, steps)], colv)
        pltpu.sync_copy(row_hbm.at[pl.ds(wid * steps, steps)], rowv)
        plsc.subcore_barrier()

        def fire(j0, bufs, sem):
            for k in range(GB):
                pltpu.async_copy(zt_hbm.at[colv.at[j0 + k]], bufs[k], sem)

        def drain_scatter(j0, bufs, sem):
            for k in range(GB):
                pltpu.make_async_copy(zt_hbm.at[colv.at[j0 + k]], bufs[k],
                                      sem).wait()
                pltpu.sync_copy(bufs[k], acc.at[rowv.at[j0 + k]], add=True)

        fire(0, bufa, gsa)

        def body(i, carry):
            ja = 2 * GB * i
            jb = ja + GB
            fire(jb, bufb, gsb)
            drain_scatter(ja, bufa, gsa)

            @pl.when(jb + GB < steps)
            def _():
                fire(jb + GB, bufa, gsa)

            drain_scatter(jb, bufb, gsb)
            return carry

        lax.fori_loop(0, steps // (2 * GB), body, 0)
        plsc.subcore_barrier()
        pltpu.sync_copy(acc.at[pl.ds(s * rows_pt, rows_pt)],
                        out_hbm.at[c, pl.ds(s * rows_pt, rows_pt)])

    return agg_kernel


def _dinv_from_hist(h_blk):
    ones = jnp.ones((NW, 1), jnp.float32)
    deg = lax.dot_general(h_blk, ones, (((0,), (0,)), ((), ())),
                          precision=lax.Precision.HIGHEST,
                          preferred_element_type=jnp.float32)
    return jnp.where(deg > 0, lax.rsqrt(jnp.maximum(deg, 1e-30)), 0.0)


def _full(shape):
    nd = len(shape)
    return pl.BlockSpec(shape, lambda: (0,) * nd)


def _tc_layer1(verts, hist, w0, w1, b1):
    n, d = verts.shape
    h1 = w0.shape[1]

    def body(v_ref, h_ref, w0_ref, w1_ref, b_ref, y_ref, zt_ref):
        dinv = _dinv_from_hist(h_ref[...])
        v = v_ref[...]
        y_ref[...] = (jnp.dot(v, w0_ref[...], preferred_element_type=jnp.float32)
                      + b_ref[...])
        zt_ref[...] = jnp.dot(v, w1_ref[...],
                              preferred_element_type=jnp.float32) * dinv

    return pl.pallas_call(
        body,
        in_specs=[_full((n, d)), _full((NW, n)), _full((d, h1)),
                  _full((d, h1)), _full((1, h1))],
        out_specs=[_full((n, h1)), _full((n, h1))],
        out_shape=[
            jax.ShapeDtypeStruct((n, h1), jnp.float32),
            jax.ShapeDtypeStruct((n, h1), jnp.float32),
        ],
    )(verts, hist, w0, w1, b1)


def _tc_layer2(y1, agg1, hist, w0, w1, b2):
    n, h1 = y1.shape
    h2 = w0.shape[1]

    def body(y_ref, a_ref, h_ref, w0_ref, w1_ref, b_ref, y2_ref, zt2_ref):
        dinv = _dinv_from_hist(h_ref[...])
        a = (a_ref[0] + a_ref[1]) * dinv
        hid = jnp.maximum(y_ref[...] - a, 0.0)
        y2_ref[...] = (jnp.dot(hid, w0_ref[...],
                               preferred_element_type=jnp.float32) + b_ref[...])
        zt2_ref[...] = jnp.dot(hid, w1_ref[...],
                               preferred_element_type=jnp.float32) * dinv

    return pl.pallas_call(
        body,
        in_specs=[_full((n, h1)), _full((NC, n, h1)), _full((NW, n)),
                  _full((h1, h2)), _full((h1, h2)), _full((1, h2))],
        out_specs=[_full((n, h2)), _full((n, h2))],
        out_shape=[
            jax.ShapeDtypeStruct((n, h2), jnp.float32),
            jax.ShapeDtypeStruct((n, h2), jnp.float32),
        ],
    )(y1, agg1, hist, w0, w1, b2)


def _tc_final(y2, agg2, hist):
    n, h2 = y2.shape

    def body(y_ref, a_ref, h_ref, o_ref):
        dinv = _dinv_from_hist(h_ref[...])
        o_ref[...] = y_ref[...] - (a_ref[0] + a_ref[1]) * dinv

    return pl.pallas_call(
        body,
        in_specs=[_full((n, h2)), _full((NC, n, h2)), _full((NW, n))],
        out_specs=_full((n, h2)),
        out_shape=jax.ShapeDtypeStruct((n, h2), jnp.float32),
    )(y2, agg2, hist)


def kernel(verts, edges, l1_W0, l1_W1, l1_b, l2_W0, l2_W1, l2_b):
    n, _ = verts.shape
    e = edges.shape[0]
    row = edges[:, 0]
    col = edges[:, 1]
    row2d = row.reshape(e // CH, CH)
    col2d = col.reshape(e // CH, CH)

    hist = _make_hist(n, e)(row)

    y1, zt1 = _tc_layer1(verts, hist, l1_W0, l1_W1, l1_b.reshape(1, -1))
    agg1 = _make_agg(n, e, l1_W0.shape[1])(zt1, col2d, row2d)
    y2, zt2 = _tc_layer2(y1, agg1, hist, l2_W0, l2_W1, l2_b.reshape(1, -1))
    agg2 = _make_agg(n, e, l2_W0.shape[1])(zt2, col2d, row2d)
    return _tc_final(y2, agg2, hist)
